# baseline (speedup 1.0000x reference)
import functools

import jax
import jax.numpy as jnp
import numpy as np
from jax import lax
from jax.experimental import pallas as pl
from jax.experimental.pallas import tpu as pltpu, tpu_sc as plsc

NUM_CORES = 2
NUM_SUBCORES = 16
NW = NUM_CORES * NUM_SUBCORES
LANES = 16

VOCAB = 1000000
D = 64
SEQ = 20
BATCH = 4096
ROWS = BATCH * SEQ
GATH_ROWS = ROWS + 40

CV = 512
W_SLICE = 61 * CV
HIT_CAP = 4224
SUP_CAP = 512
BKT_CAP = 128
NCHUNK_MAX = 63

_MAGIC = np.int32(0x5F3759DF)
_RMASK = np.int32(0x1FFFF)


def _rsqrt(x):
    i = plsc.bitcast(x, jnp.int32)
    i = _MAGIC - lax.shift_right_logical(i, 1)
    y = plsc.bitcast(i, jnp.float32)
    neg_half_x = x * np.float32(-0.5)
    for _ in range(3):
        y = y * (neg_half_x * y * y + np.float32(1.5))
    return y


def _k1_body(embT_ref, tokT_ref, gath_ref,
             sbuf, hits, sup, bkt, cbuf, stage, cnt_s, gsem, rsem):
    wid = lax.axis_index("s") * NUM_CORES + lax.axis_index("c")
    w_lo = wid * W_SLICE
    w_hi = jnp.where(wid == NW - 1, np.int32(VOCAB),
                     (wid + 1) * W_SLICE).astype(jnp.int32)
    nchunks = jnp.where(wid == NW - 1, np.int32(NCHUNK_MAX), np.int32(61))
    iota = lax.iota(jnp.int32, LANES)

    ptr = np.int32(0)
    for l0, nl in ((0, 8), (8, 8), (16, 4)):
        for bh in range(2):
            pltpu.sync_copy(
                tokT_ref.at[pl.ds(l0, nl), pl.ds(bh * 2048, 2048)],
                sbuf.at[pl.ds(0, nl), :])

            def scan_body(ci, p, l0=l0, nl=nl, bh=bh):
                for lr in range(nl):
                    tv = sbuf[lr, pl.ds(ci * LANES, LANES)]
                    m = (tv >= w_lo) & (tv < w_hi)
                    vl = tv - w_lo
                    r = ((l0 + lr) * 4096 + bh * 2048 + ci * LANES) + iota
                    packed = lax.shift_left(vl, 17) | r
                    plsc.store_compressed(hits.at[pl.ds(p, LANES)], packed, mask=m)
                    n = plsc.all_reduce_population_count(m)
                    p = p + n[0]
                return p

            ptr = lax.fori_loop(0, 2048 // LANES, scan_body, ptr)

    ng = lax.shift_right_logical(ptr + (LANES - 1), 4)
    for s in range(8):
        def sup_body(g, sp, s=s):
            pv = hits[pl.ds(g * LANES, LANES)]
            valid = (g * LANES + iota) < ptr
            m = valid & (lax.shift_right_logical(pv, 29) == np.int32(s))
            plsc.store_compressed(
                sup.at[pl.ds(s * SUP_CAP + sp, LANES)], pv, mask=m)
            n = plsc.all_reduce_population_count(m)
            return sp + n[0]

        sp = lax.fori_loop(0, ng, sup_body, np.int32(0))
        ngs = lax.shift_right_logical(sp + (LANES - 1), 4)
        for c8 in range(8):
            ci = s * 8 + c8
            if ci >= NCHUNK_MAX:
                continue

            def bkt_body(g, bp, s=s, ci=ci):
                pv = sup[pl.ds(s * SUP_CAP + g * LANES, LANES)]
                valid = (g * LANES + iota) < sp
                m = valid & (
                    lax.shift_right_logical(pv, 26) == np.int32(ci))
                plsc.store_compressed(bkt.at[ci, pl.ds(bp, LANES)], pv, mask=m)
                n = plsc.all_reduce_population_count(m)
                return bp + n[0]

            bp = lax.fori_loop(0, ngs, bkt_body, np.int32(0))
            dummy = np.uint32(((ci * CV) << 17) | ROWS).astype(np.int32)
            bkt[ci, pl.ds(bp, LANES)] = iota * np.int32(0) + dummy
            cnt_s[ci] = bp

    jidx = [iota + np.int32(m * LANES) for m in range(D // LANES)]

    def fire_chunk(i):
        b = lax.rem(i, np.int32(2))
        v0 = w_lo + i * CV

        @pl.when(i != NCHUNK_MAX - 1)
        def _():
            for jb in range(8):
                pltpu.async_copy(
                    embT_ref.at[pl.ds(jb * 8, 8), pl.ds(v0, CV)],
                    cbuf.at[b, pl.ds(jb * 8, 8), :],
                    gsem.at[b])

        @pl.when(i == NCHUNK_MAX - 1)
        def _():
            for jb in range(8):
                pltpu.async_copy(
                    embT_ref.at[pl.ds(jb * 8, 8), pl.ds(v0, 64)],
                    cbuf.at[b, pl.ds(jb * 8, 8), pl.ds(0, 64)],
                    gsem.at[b])

    def drain_chunk(i):
        b = lax.rem(i, np.int32(2))

        @pl.when(i != NCHUNK_MAX - 1)
        def _():
            for jb in range(8):
                pltpu.make_async_copy(
                    embT_ref.at[pl.ds(0, 8), pl.ds(0, CV)],
                    cbuf.at[b, pl.ds(0, 8), :],
                    gsem.at[b]).wait()

        @pl.when(i == NCHUNK_MAX - 1)
        def _():
            for jb in range(8):
                pltpu.make_async_copy(
                    embT_ref.at[pl.ds(0, 8), pl.ds(0, 64)],
                    cbuf.at[b, pl.ds(0, 8), pl.ds(0, 64)],
                    gsem.at[b]).wait()

    fire_chunk(np.int32(0))
    for k in range(LANES):
        pltpu.async_copy(stage.at[k], gath_ref.at[ROWS], rsem.at[k])

    def chunk_body(i, carry):
        b = lax.rem(i, np.int32(2))
        bsplat = lax.broadcast(b, (LANES,))
        drain_chunk(i)

        @pl.when(i + 1 < nchunks)
        def _():
            fire_chunk(i + 1)

        n = cnt_s[i]

        v0_local = i * CV

        def ebody(g, carry2):
            pv = bkt[i, pl.ds(g * LANES, LANES)]
            for k in range(LANES):
                pk = pv[k]
                vl = lax.shift_right_logical(pk, 17) - v0_local
                r = pk & _RMASK
                vsplat = lax.broadcast(vl, (LANES,))
                pltpu.make_async_copy(
                    stage.at[k], gath_ref.at[ROWS], rsem.at[k]).wait()
                for m in range(D // LANES):
                    gm = plsc.load_gather(cbuf, [bsplat, jidx[m], vsplat])
                    stage[k, pl.ds(m * LANES, LANES)] = gm
                pltpu.async_copy(stage.at[k], gath_ref.at[r], rsem.at[k])
            return carry2

        lax.fori_loop(0, lax.shift_right_logical(n + (LANES - 1), 4),
                      ebody, carry)
        return carry

    lax.fori_loop(0, nchunks, chunk_body, np.int32(0))
    for k in range(LANES):
        pltpu.make_async_copy(
            stage.at[k], gath_ref.at[ROWS], rsem.at[k]).wait()

    def csum(ci, acc):
        return acc + cnt_s[ci]

    total_bkt = lax.fori_loop(0, NCHUNK_MAX, csum, np.int32(0))
    dv = lax.broadcast(np.float32(0.0), (LANES,))
    dv = jnp.where(iota == 0, ptr.astype(jnp.float32), dv)
    dv = jnp.where(iota == 1, total_bkt.astype(jnp.float32), dv)
    stage[0, pl.ds(0, LANES)] = dv
    pltpu.async_copy(stage.at[0], gath_ref.at[ROWS + 8 + wid],
                     rsem.at[0]).wait()


def _k2_body(gath_ref, pos_ref, gam_ref, bet_ref, out_ref,
             buf, stage, pos_v, gam_v, bet_v, gsem, osem):
    wid = lax.axis_index("s") * NUM_CORES + lax.axis_index("c")
    b0 = wid * 128
    iota = lax.iota(jnp.int32, LANES)

    pltpu.sync_copy(pos_ref, pos_v)
    pltpu.sync_copy(gam_ref, gam_v)
    pltpu.sync_copy(bet_ref, bet_v)
    gvec = [gam_v[pl.ds(j * LANES, LANES)] for j in range(D // LANES)]
    bvec = [bet_v[pl.ds(j * LANES, LANES)] for j in range(D // LANES)]

    def fire(l):
        b = lax.rem(l, np.int32(2))
        pltpu.async_copy(
            gath_ref.at[pl.ds(l * BATCH + b0, 128), :],
            buf.at[b], gsem.at[b])

    fire(np.int32(0))

    def lbody(l, carry):
        b = lax.rem(l, np.int32(2))
        bsplat = lax.broadcast(b, (LANES,))
        pltpu.make_async_copy(
            gath_ref.at[pl.ds(0, 128), :], buf.at[b], gsem.at[b]).wait()

        @pl.when(l + 1 < SEQ)
        def _():
            fire(l + 1)

        @pl.when(l >= 2)
        def _():
            pltpu.make_async_copy(
                stage.at[b], out_ref.at[0, :, pl.ds(0, 128)],
                osem.at[b]).wait()

        pvec = [pos_v[l, pl.ds(j * LANES, LANES)] for j in range(D // LANES)]

        def row_body(rr, carry2):
            nj = D // LANES
            x = [buf[b, rr, pl.ds(j * LANES, LANES)] + pvec[j]
                 for j in range(nj)]
            ssum = x[0] + x[1] + x[2] + x[3]
            mean = lax.broadcast(jnp.sum(ssum), (LANES,)) * np.float32(1.0 / D)
            d = [xj - mean for xj in x]
            sq = d[0] * d[0] + d[1] * d[1] + d[2] * d[2] + d[3] * d[3]
            var = lax.broadcast(jnp.sum(sq), (LANES,)) * np.float32(1.0 / D)
            rinv = _rsqrt(var + np.float32(1e-5))
            for j in range(nj):
                buf[b, rr, pl.ds(j * LANES, LANES)] = (
                    d[j] * (rinv * gvec[j]) + bvec[j])
            return carry2

        lax.fori_loop(0, 128, row_body, np.int32(0))

        def tbody2(bg, carry2):
            bidx = bg * LANES + iota
            for j in range(D):
                gm = plsc.load_gather(
                    buf, [bsplat, bidx, lax.broadcast(np.int32(j), (LANES,))])
                stage[b, j, pl.ds(bg * LANES, LANES)] = gm
            return carry2

        lax.fori_loop(0, 128 // LANES, tbody2, np.int32(0))

        pltpu.async_copy(
            stage.at[b], out_ref.at[l, :, pl.ds(b0, 128)], osem.at[b])
        return carry

    lax.fori_loop(0, SEQ, lbody, np.int32(0))
    for sb in range(2):
        pltpu.make_async_copy(
            stage.at[sb], out_ref.at[0, :, pl.ds(0, 128)],
            osem.at[sb]).wait()


_SC_MESH = plsc.VectorSubcoreMesh(
    core_axis_name="c", subcore_axis_name="s",
    num_cores=NUM_CORES, num_subcores=NUM_SUBCORES)
_SC_PARAMS = pltpu.CompilerParams(
    needs_layout_passes=False, use_tc_tiling_on_sc=True)


_k1 = functools.partial(
    pl.kernel,
    out_type=jax.ShapeDtypeStruct((GATH_ROWS, D), jnp.float32),
    mesh=_SC_MESH,
    scratch_types=[
        pltpu.VMEM((8, 2048), jnp.int32),
        pltpu.VMEM((HIT_CAP,), jnp.int32),
        pltpu.VMEM((8 * SUP_CAP,), jnp.int32),
        pltpu.VMEM((NCHUNK_MAX, BKT_CAP), jnp.int32),
        pltpu.VMEM((2, D, CV), jnp.float32),
        pltpu.VMEM((LANES, D), jnp.float32),
        pltpu.SMEM((NCHUNK_MAX + 1,), jnp.int32),
        pltpu.SemaphoreType.DMA((2,)),
        pltpu.SemaphoreType.DMA((LANES,)),
    ],
    compiler_params=_SC_PARAMS,
)(_k1_body)


_k2 = functools.partial(
    pl.kernel,
    out_type=jax.ShapeDtypeStruct((SEQ, D, BATCH), jnp.float32),
    mesh=_SC_MESH,
    scratch_types=[
        pltpu.VMEM((2, 128, D), jnp.float32),
        pltpu.VMEM((2, D, 128), jnp.float32),
        pltpu.VMEM((SEQ, D), jnp.float32),
        pltpu.VMEM((D,), jnp.float32),
        pltpu.VMEM((D,), jnp.float32),
        pltpu.SemaphoreType.DMA((2,)),
        pltpu.SemaphoreType.DMA((2,)),
    ],
    compiler_params=_SC_PARAMS,
)(_k2_body)


@jax.jit
def kernel(tokens, embedding, pos_embedding, ln_gamma, ln_beta):
    B, L = tokens.shape
    tokT = tokens.astype(jnp.int32).T
    embT = embedding.T
    gath = _k1(embT, tokT)
    out = _k2(gath, pos_embedding, ln_gamma, ln_beta)
    return jnp.transpose(out, (2, 0, 1))


# speedup vs baseline: 1.1165x; 1.1165x over previous
import functools

import jax
import jax.numpy as jnp
import numpy as np
from jax import lax
from jax.experimental import pallas as pl
from jax.experimental.pallas import tpu as pltpu, tpu_sc as plsc

NUM_CORES = 2
NUM_SUBCORES = 16
NW = NUM_CORES * NUM_SUBCORES
LANES = 16

VOCAB = 1000000
D = 64
SEQ = 20
BATCH = 4096
ROWS = BATCH * SEQ
GATH_ROWS = ROWS + 40

CV = 512
W_SLICE = 61 * CV
HIT_CAP = 4224
SUP_CAP = 512
BKT_CAP = 128
NCHUNK_MAX = 63

_MAGIC = np.int32(0x5F3759DF)
_RMASK = np.int32(0x1FFFF)


def _rsqrt(x):
    i = plsc.bitcast(x, jnp.int32)
    i = _MAGIC - lax.shift_right_logical(i, 1)
    y = plsc.bitcast(i, jnp.float32)
    neg_half_x = x * np.float32(-0.5)
    for _ in range(3):
        y = y * (neg_half_x * y * y + np.float32(1.5))
    return y


def _k1_body(embT_ref, tokT_ref, gath_ref,
             sbuf, hits, sup, bkt, cbuf, stage, cnt_s, gsem, rsem):
    wid = lax.axis_index("s") * NUM_CORES + lax.axis_index("c")
    w_lo = wid * W_SLICE
    w_hi = jnp.where(wid == NW - 1, np.int32(VOCAB),
                     (wid + 1) * W_SLICE).astype(jnp.int32)
    nchunks = jnp.where(wid == NW - 1, np.int32(NCHUNK_MAX), np.int32(61))
    iota = lax.iota(jnp.int32, LANES)

    ptr = np.int32(0)
    for l0, nl in ((0, 8), (8, 8), (16, 4)):
        for bh in range(2):
            pltpu.sync_copy(
                tokT_ref.at[pl.ds(l0, nl), pl.ds(bh * 2048, 2048)],
                sbuf.at[pl.ds(0, nl), :])

            def scan_body(ci, p, l0=l0, nl=nl, bh=bh):
                ms, packeds, ns = [], [], []
                for lr in range(nl):
                    tv = sbuf[lr, pl.ds(ci * LANES, LANES)]
                    m = (tv >= w_lo) & (tv < w_hi)
                    vl = tv - w_lo
                    r = ((l0 + lr) * 4096 + bh * 2048 + ci * LANES) + iota
                    ms.append(m)
                    packeds.append(lax.shift_left(vl, 17) | r)
                    ns.append(plsc.all_reduce_population_count(m))
                for lr in range(nl):
                    plsc.store_compressed(
                        hits.at[pl.ds(p, LANES)], packeds[lr], mask=ms[lr])
                    p = p + ns[lr][0]
                return p

            ptr = lax.fori_loop(0, 2048 // LANES, scan_body, ptr)

    ng = lax.shift_right_logical(ptr + (LANES - 1), 4)
    for s in range(8):
        def sup_body(g, sp, s=s):
            pv = hits[pl.ds(g * LANES, LANES)]
            valid = (g * LANES + iota) < ptr
            m = valid & (lax.shift_right_logical(pv, 29) == np.int32(s))
            plsc.store_compressed(
                sup.at[pl.ds(s * SUP_CAP + sp, LANES)], pv, mask=m)
            n = plsc.all_reduce_population_count(m)
            return sp + n[0]

        sp = lax.fori_loop(0, ng, sup_body, np.int32(0))
        ngs = lax.shift_right_logical(sp + (LANES - 1), 4)
        for c8 in range(8):
            ci = s * 8 + c8
            if ci >= NCHUNK_MAX:
                continue

            def bkt_body(g, bp, s=s, ci=ci):
                pv = sup[pl.ds(s * SUP_CAP + g * LANES, LANES)]
                valid = (g * LANES + iota) < sp
                m = valid & (
                    lax.shift_right_logical(pv, 26) == np.int32(ci))
                plsc.store_compressed(bkt.at[ci, pl.ds(bp, LANES)], pv, mask=m)
                n = plsc.all_reduce_population_count(m)
                return bp + n[0]

            bp = lax.fori_loop(0, ngs, bkt_body, np.int32(0))
            dummy = np.uint32(((ci * CV) << 17) | ROWS).astype(np.int32)
            bkt[ci, pl.ds(bp, LANES)] = iota * np.int32(0) + dummy
            cnt_s[ci] = bp

    jidx = [iota + np.int32(m * LANES) for m in range(D // LANES)]

    def fire_chunk(i):
        b = lax.rem(i, np.int32(2))
        v0 = w_lo + i * CV

        @pl.when(i != NCHUNK_MAX - 1)
        def _():
            for jb in range(8):
                pltpu.async_copy(
                    embT_ref.at[pl.ds(jb * 8, 8), pl.ds(v0, CV)],
                    cbuf.at[b, pl.ds(jb * 8, 8), :],
                    gsem.at[b])

        @pl.when(i == NCHUNK_MAX - 1)
        def _():
            for jb in range(8):
                pltpu.async_copy(
                    embT_ref.at[pl.ds(jb * 8, 8), pl.ds(v0, 64)],
                    cbuf.at[b, pl.ds(jb * 8, 8), pl.ds(0, 64)],
                    gsem.at[b])

    def drain_chunk(i):
        b = lax.rem(i, np.int32(2))

        @pl.when(i != NCHUNK_MAX - 1)
        def _():
            for jb in range(8):
                pltpu.make_async_copy(
                    embT_ref.at[pl.ds(0, 8), pl.ds(0, CV)],
                    cbuf.at[b, pl.ds(0, 8), :],
                    gsem.at[b]).wait()

        @pl.when(i == NCHUNK_MAX - 1)
        def _():
            for jb in range(8):
                pltpu.make_async_copy(
                    embT_ref.at[pl.ds(0, 8), pl.ds(0, 64)],
                    cbuf.at[b, pl.ds(0, 8), pl.ds(0, 64)],
                    gsem.at[b]).wait()

    fire_chunk(np.int32(0))
    for k in range(LANES):
        pltpu.async_copy(stage.at[k], gath_ref.at[ROWS], rsem.at[k])

    def chunk_body(i, carry):
        b = lax.rem(i, np.int32(2))
        bsplat = lax.broadcast(b, (LANES,))

        @pl.when(i + 1 < nchunks)
        def _():
            fire_chunk(i + 1)

        drain_chunk(i)

        n = cnt_s[i]

        v0_local = i * CV

        def ebody(g, carry2):
            pv = bkt[i, pl.ds(g * LANES, LANES)]
            for k in range(LANES):
                pk = pv[k]
                vl = lax.shift_right_logical(pk, 17) - v0_local
                r = pk & _RMASK
                vsplat = lax.broadcast(vl, (LANES,))
                pltpu.make_async_copy(
                    stage.at[k], gath_ref.at[ROWS], rsem.at[k]).wait()
                for m in range(D // LANES):
                    gm = plsc.load_gather(cbuf, [bsplat, jidx[m], vsplat])
                    stage[k, pl.ds(m * LANES, LANES)] = gm
                pltpu.async_copy(stage.at[k], gath_ref.at[r], rsem.at[k])
            return carry2

        lax.fori_loop(0, lax.shift_right_logical(n + (LANES - 1), 4),
                      ebody, carry)
        return carry

    lax.fori_loop(0, nchunks, chunk_body, np.int32(0))
    for k in range(LANES):
        pltpu.make_async_copy(
            stage.at[k], gath_ref.at[ROWS], rsem.at[k]).wait()


def _k2_body(gath_ref, pos_ref, gam_ref, bet_ref, out_ref,
             buf, stage, pos_v, gam_v, bet_v, gsem, osem):
    wid = lax.axis_index("s") * NUM_CORES + lax.axis_index("c")
    b0 = wid * 128
    iota = lax.iota(jnp.int32, LANES)

    pltpu.sync_copy(pos_ref, pos_v)
    pltpu.sync_copy(gam_ref, gam_v)
    pltpu.sync_copy(bet_ref, bet_v)
    gvec = [gam_v[pl.ds(j * LANES, LANES)] for j in range(D // LANES)]
    bvec = [bet_v[pl.ds(j * LANES, LANES)] for j in range(D // LANES)]

    def fire(l):
        b = lax.rem(l, np.int32(2))
        pltpu.async_copy(
            gath_ref.at[pl.ds(l * BATCH + b0, 128), :],
            buf.at[b], gsem.at[b])

    fire(np.int32(0))

    def lbody(l, carry):
        b = lax.rem(l, np.int32(2))
        bsplat = lax.broadcast(b, (LANES,))
        pltpu.make_async_copy(
            gath_ref.at[pl.ds(0, 128), :], buf.at[b], gsem.at[b]).wait()

        @pl.when(l + 1 < SEQ)
        def _():
            fire(l + 1)

        @pl.when(l >= 2)
        def _():
            pltpu.make_async_copy(
                stage.at[b], out_ref.at[0, :, pl.ds(0, 128)],
                osem.at[b]).wait()

        pvec = [pos_v[l, pl.ds(j * LANES, LANES)] for j in range(D // LANES)]

        def row_body(rr, carry2):
            nj = D // LANES
            x = [buf[b, rr, pl.ds(j * LANES, LANES)] + pvec[j]
                 for j in range(nj)]
            ssum = x[0] + x[1] + x[2] + x[3]
            mean = lax.broadcast(jnp.sum(ssum), (LANES,)) * np.float32(1.0 / D)
            d = [xj - mean for xj in x]
            sq = d[0] * d[0] + d[1] * d[1] + d[2] * d[2] + d[3] * d[3]
            var = lax.broadcast(jnp.sum(sq), (LANES,)) * np.float32(1.0 / D)
            rinv = _rsqrt(var + np.float32(1e-5))
            for j in range(nj):
                buf[b, rr, pl.ds(j * LANES, LANES)] = (
                    d[j] * (rinv * gvec[j]) + bvec[j])
            return carry2

        lax.fori_loop(0, 128, row_body, np.int32(0), unroll=4)

        def tbody2(bg, carry2):
            bidx = bg * LANES + iota
            for j in range(D):
                gm = plsc.load_gather(
                    buf, [bsplat, bidx, lax.broadcast(np.int32(j), (LANES,))])
                stage[b, j, pl.ds(bg * LANES, LANES)] = gm
            return carry2

        lax.fori_loop(0, 128 // LANES, tbody2, np.int32(0))

        pltpu.async_copy(
            stage.at[b], out_ref.at[l, :, pl.ds(b0, 128)], osem.at[b])
        return carry

    lax.fori_loop(0, SEQ, lbody, np.int32(0))
    for sb in range(2):
        pltpu.make_async_copy(
            stage.at[sb], out_ref.at[0, :, pl.ds(0, 128)],
            osem.at[sb]).wait()


_SC_MESH = plsc.VectorSubcoreMesh(
    core_axis_name="c", subcore_axis_name="s",
    num_cores=NUM_CORES, num_subcores=NUM_SUBCORES)
_SC_PARAMS = pltpu.CompilerParams(
    needs_layout_passes=False, use_tc_tiling_on_sc=True)


_k1 = functools.partial(
    pl.kernel,
    out_type=jax.ShapeDtypeStruct((GATH_ROWS, D), jnp.float32),
    mesh=_SC_MESH,
    scratch_types=[
        pltpu.VMEM((8, 2048), jnp.int32),
        pltpu.VMEM((HIT_CAP,), jnp.int32),
        pltpu.VMEM((8 * SUP_CAP,), jnp.int32),
        pltpu.VMEM((NCHUNK_MAX, BKT_CAP), jnp.int32),
        pltpu.VMEM((2, D, CV), jnp.float32),
        pltpu.VMEM((LANES, D), jnp.float32),
        pltpu.SMEM((NCHUNK_MAX + 1,), jnp.int32),
        pltpu.SemaphoreType.DMA((2,)),
        pltpu.SemaphoreType.DMA((LANES,)),
    ],
    compiler_params=_SC_PARAMS,
)(_k1_body)


_k2 = functools.partial(
    pl.kernel,
    out_type=jax.ShapeDtypeStruct((SEQ, D, BATCH), jnp.float32),
    mesh=_SC_MESH,
    scratch_types=[
        pltpu.VMEM((2, 128, D), jnp.float32),
        pltpu.VMEM((2, D, 128), jnp.float32),
        pltpu.VMEM((SEQ, D), jnp.float32),
        pltpu.VMEM((D,), jnp.float32),
        pltpu.VMEM((D,), jnp.float32),
        pltpu.SemaphoreType.DMA((2,)),
        pltpu.SemaphoreType.DMA((2,)),
    ],
    compiler_params=_SC_PARAMS,
)(_k2_body)


@jax.jit
def kernel(tokens, embedding, pos_embedding, ln_gamma, ln_beta):
    B, L = tokens.shape
    tokT = tokens.astype(jnp.int32).T
    embT = embedding.T
    gath = _k1(embT, tokT)
    out = _k2(gath, pos_embedding, ln_gamma, ln_beta)
    return jnp.transpose(out, (2, 0, 1))


# speedup vs baseline: 2.1320x; 1.9096x over previous
import functools

import jax
import jax.numpy as jnp
import numpy as np
from jax import lax
from jax.experimental import pallas as pl
from jax.experimental.pallas import tpu as pltpu, tpu_sc as plsc

NUM_CORES = 2
NUM_SUBCORES = 16
NW = NUM_CORES * NUM_SUBCORES
LANES = 16

VOCAB = 1000000
D = 64
SEQ = 20
BATCH = 4096
ROWS = BATCH * SEQ
PER_W = ROWS // NW
CHUNK = 320
CHUNKS = PER_W // CHUNK

_MAGIC = np.int32(0x5F3759DF)


def _rsqrt(x):
    i = plsc.bitcast(x, jnp.int32)
    i = _MAGIC - lax.shift_right_logical(i, 1)
    y = plsc.bitcast(i, jnp.float32)
    neg_half_x = x * np.float32(-0.5)
    for _ in range(3):
        y = y * (neg_half_x * y * y + np.float32(1.5))
    return y


def _ln_row(rows_v, b, row, pos_v, p, gvec, bvec):
    nj = D // LANES
    x = [rows_v[b, row, pl.ds(j * LANES, LANES)]
         + pos_v[p, pl.ds(j * LANES, LANES)]
         for j in range(nj)]
    s = x[0] + x[1] + x[2] + x[3]
    mean = lax.broadcast(jnp.sum(s), (LANES,)) * np.float32(1.0 / D)
    d = [xj - mean for xj in x]
    sq = d[0] * d[0] + d[1] * d[1] + d[2] * d[2] + d[3] * d[3]
    var = lax.broadcast(jnp.sum(sq), (LANES,)) * np.float32(1.0 / D)
    rinv = _rsqrt(var + np.float32(1e-5))
    for j in range(nj):
        rows_v[b, row, pl.ds(j * LANES, LANES)] = (
            d[j] * (rinv * gvec[j]) + bvec[j])


def _body(tok_ref, emb_ref, pos_ref, gam_ref, bet_ref, out_ref,
          idx_v, rows_v, pos_v, gam_v, bet_v, gsem):
    wid = lax.axis_index("s") * NUM_CORES + lax.axis_index("c")
    base = wid * PER_W

    pltpu.sync_copy(tok_ref.at[wid], idx_v)
    pltpu.sync_copy(pos_ref, pos_v)
    pltpu.sync_copy(gam_ref, gam_v)
    pltpu.sync_copy(bet_ref, bet_v)

    def fire_gather(c):
        b = lax.rem(c, np.int32(2))

        def enqueue(g, carry):
            tv = idx_v[pl.ds(c * CHUNK + g * LANES, LANES)]
            for k in range(LANES):
                pltpu.async_copy(
                    emb_ref.at[tv[k]],
                    rows_v.at[b, g * LANES + k],
                    gsem.at[b],
                )
            return carry

        lax.fori_loop(0, CHUNK // LANES, enqueue, np.int32(0))

    def drain_gather(c):
        b = lax.rem(c, np.int32(2))

        def drain(r, carry):
            pltpu.make_async_copy(
                emb_ref.at[0], rows_v.at[b, r], gsem.at[b]).wait()
            return carry

        lax.fori_loop(0, CHUNK, drain, np.int32(0))

    gvec = [gam_v[pl.ds(j * LANES, LANES)] for j in range(D // LANES)]
    bvec = [bet_v[pl.ds(j * LANES, LANES)] for j in range(D // LANES)]

    fire_gather(np.int32(0))

    def chunk_body(c, carry):
        b = lax.rem(c, np.int32(2))
        drain_gather(c)

        @pl.when(c + 1 < CHUNKS)
        def _():
            fire_gather(c + 1)

        def row_body(r, carry2):
            p = lax.rem(r, np.int32(SEQ))
            _ln_row(rows_v, b, r, pos_v, p, gvec, bvec)
            return carry2

        lax.fori_loop(0, CHUNK, row_body, np.int32(0), unroll=4)

        pltpu.sync_copy(
            rows_v.at[b], out_ref.at[pl.ds(base + c * CHUNK, CHUNK)])
        return carry

    lax.fori_loop(0, CHUNKS, chunk_body, np.int32(0))


@functools.partial(
    pl.kernel,
    out_type=jax.ShapeDtypeStruct((ROWS, D), jnp.float32),
    mesh=plsc.VectorSubcoreMesh(
        core_axis_name="c", subcore_axis_name="s",
        num_cores=NUM_CORES, num_subcores=NUM_SUBCORES),
    scratch_types=[
        pltpu.VMEM((PER_W,), jnp.int32),
        pltpu.VMEM((2, CHUNK, D), jnp.float32),
        pltpu.VMEM((SEQ, D), jnp.float32),
        pltpu.VMEM((D,), jnp.float32),
        pltpu.VMEM((D,), jnp.float32),
        pltpu.SemaphoreType.DMA((2,)),
    ],
    compiler_params=pltpu.CompilerParams(
        needs_layout_passes=False, use_tc_tiling_on_sc=True),
)
def _encoder_sc(tok_ref, emb_ref, pos_ref, gam_ref, bet_ref, out_ref,
                idx_v, rows_v, pos_v, gam_v, bet_v, gsem):
    _body(tok_ref, emb_ref, pos_ref, gam_ref, bet_ref, out_ref,
          idx_v, rows_v, pos_v, gam_v, bet_v, gsem)


@jax.jit
def kernel(tokens, embedding, pos_embedding, ln_gamma, ln_beta):
    B, L = tokens.shape
    tok = tokens.astype(jnp.int32).reshape(NW, PER_W)
    out = _encoder_sc(tok, embedding, pos_embedding, ln_gamma, ln_beta)
    return out.reshape(B, L, D)
